# baseline (speedup 1.0000x reference)
import functools

import jax
import jax.numpy as jnp
from jax import lax
from jax.experimental import pallas as pl
from jax.experimental.pallas import tpu as pltpu
from jax.experimental.pallas import tpu_sc as plsc

NUM_EMBEDDINGS = 1000000
EMBEDDING_DIM = 64
BATCH = 16384
HIST = 50

_TOTAL = BATCH * HIST
_CHUNK = 128
_NBUF = 8

_TBLK = 4096
_BBLK = 4096


def _table_to_row_major(emb_t):
    grid = (NUM_EMBEDDINGS + _TBLK - 1) // _TBLK

    def body(in_ref, out_ref):
        out_ref[...] = in_ref[...].T

    return pl.pallas_call(
        body,
        grid=(grid,),
        in_specs=[pl.BlockSpec((EMBEDDING_DIM, _TBLK), lambda i: (0, i))],
        out_specs=pl.BlockSpec((_TBLK, EMBEDDING_DIM), lambda i: (i, 0)),
        out_shape=jax.ShapeDtypeStruct((NUM_EMBEDDINGS, EMBEDDING_DIM), jnp.float32),
    )(emb_t)


def _rows_to_batch_minor(flat_hm):
    per_h = BATCH // _BBLK

    def body(in_ref, out_ref):
        out_ref[0] = in_ref[...].T

    return pl.pallas_call(
        body,
        grid=(HIST, per_h),
        in_specs=[
            pl.BlockSpec((_BBLK, EMBEDDING_DIM), lambda h, i: (h * per_h + i, 0))
        ],
        out_specs=pl.BlockSpec((1, EMBEDDING_DIM, _BBLK), lambda h, i: (h, 0, i)),
        out_shape=jax.ShapeDtypeStruct((HIST, EMBEDDING_DIM, BATCH), jnp.float32),
    )(flat_hm)


def _make_gather(num_workers: int, num_cores: int):
    cols_per_w = BATCH // num_workers
    chunks_per_h = cols_per_w // _CHUNK
    n_chunks = HIST * chunks_per_h
    outer = n_chunks // _NBUF
    mesh = plsc.VectorSubcoreMesh(core_axis_name="c", subcore_axis_name="s")

    @functools.partial(
        pl.kernel,
        mesh=mesh,
        out_type=jax.ShapeDtypeStruct((_TOTAL, EMBEDDING_DIM), jnp.float32),
        scratch_types=[
            pltpu.VMEM((HIST, cols_per_w), jnp.int32),
            pltpu.VMEM((_NBUF, _CHUNK, EMBEDDING_DIM), jnp.float32),
            [pltpu.SemaphoreType.DMA] * _NBUF,
        ],
        compiler_params=pltpu.CompilerParams(use_tc_tiling_on_sc=False),
    )
    def gather_kernel(idx_hbm, table_hbm, out_hbm, idx_v, rows_v, sems):
        wid = lax.axis_index("s") * num_cores + lax.axis_index("c")
        col0 = wid * cols_per_w
        pltpu.sync_copy(idx_hbm.at[:, pl.ds(col0, cols_per_w)], idx_v)

        def fire(j, b):
            h = j // chunks_per_h
            k = j % chunks_per_h
            pltpu.async_copy(
                table_hbm.at[idx_v.at[h, pl.ds(k * _CHUNK, _CHUNK)]],
                rows_v.at[b],
                sems[b],
            )

        def drain(j, b):
            h = j // chunks_per_h
            k = j % chunks_per_h
            pltpu.make_async_copy(
                table_hbm.at[idx_v.at[h, pl.ds(k * _CHUNK, _CHUNK)]],
                rows_v.at[b],
                sems[b],
            ).wait()

        for b in range(_NBUF):
            fire(b, b)

        def body(j2, carry):
            for b in range(_NBUF):
                j = j2 * _NBUF + b
                drain(j, b)
                h = j // chunks_per_h
                k = j % chunks_per_h
                pltpu.sync_copy(
                    rows_v.at[b],
                    out_hbm.at[
                        pl.ds(h * BATCH + col0 + k * _CHUNK, _CHUNK)
                    ],
                )

                @pl.when(j2 + 1 < outer)
                def _():
                    fire(j + _NBUF, b)

            return carry

        lax.fori_loop(0, outer, body, 0)

    return gather_kernel


def kernel(token_ids, embedding):
    info = plsc.get_sparse_core_info()
    num_workers = info.num_cores * info.num_subcores
    tok_t = jnp.transpose(token_ids).astype(jnp.int32)
    table_rm = _table_to_row_major(jnp.transpose(embedding))
    flat_hm = _make_gather(num_workers, info.num_cores)(tok_t, table_rm)
    out_bm = _rows_to_batch_minor(flat_hm)
    return jnp.transpose(out_bm, (2, 0, 1))


# speedup vs baseline: 2.1897x; 2.1897x over previous
import functools

import jax
import jax.numpy as jnp
from jax import lax
from jax.experimental import pallas as pl
from jax.experimental.pallas import tpu as pltpu
from jax.experimental.pallas import tpu_sc as plsc

NUM_EMBEDDINGS = 1000000
EMBEDDING_DIM = 64
BATCH = 16384
HIST = 50

_TOTAL = BATCH * HIST
_CHUNK = 128
_NBUF = 8

_TBLK = 4096
_NTBLK = (NUM_EMBEDDINGS + _TBLK - 1) // _TBLK
_NROWS = _NTBLK * _TBLK
_HHALF = BATCH // 2


def _table_to_row_major(emb_t):
    def body(in_ref, out_ref):
        x = in_ref[...]
        out_ref[...] = jnp.concatenate(
            [x[:, :_TBLK // 2].T, x[:, _TBLK // 2:].T], axis=1
        )

    return pl.pallas_call(
        body,
        grid=(_NTBLK,),
        in_specs=[pl.BlockSpec((EMBEDDING_DIM, _TBLK), lambda i: (0, i))],
        out_specs=pl.BlockSpec((_TBLK // 2, 2 * EMBEDDING_DIM), lambda i: (i, 0)),
        out_shape=jax.ShapeDtypeStruct(
            (_NROWS // 2, 2 * EMBEDDING_DIM), jnp.float32
        ),
    )(emb_t)


def _rows_to_batch_minor(flat_pairs):
    def body(in_ref, out_ref):
        y = in_ref[...]
        out_ref[0] = jnp.concatenate(
            [y[:, :EMBEDDING_DIM].T, y[:, EMBEDDING_DIM:].T], axis=1
        )

    return pl.pallas_call(
        body,
        grid=(HIST,),
        in_specs=[pl.BlockSpec((_HHALF, 2 * EMBEDDING_DIM), lambda h: (h, 0))],
        out_specs=pl.BlockSpec((1, EMBEDDING_DIM, BATCH), lambda h: (h, 0, 0)),
        out_shape=jax.ShapeDtypeStruct((HIST, EMBEDDING_DIM, BATCH), jnp.float32),
    )(flat_pairs)


def _make_gather(num_workers: int, num_cores: int):
    cols_per_w = BATCH // num_workers
    chunks_per_h = cols_per_w // _CHUNK
    n_chunks = HIST * chunks_per_h
    outer = n_chunks // _NBUF
    mesh = plsc.VectorSubcoreMesh(core_axis_name="c", subcore_axis_name="s")

    @functools.partial(
        pl.kernel,
        mesh=mesh,
        out_type=jax.ShapeDtypeStruct((HIST * _HHALF, 2 * EMBEDDING_DIM),
                                      jnp.float32),
        scratch_types=[
            pltpu.VMEM((HIST, cols_per_w), jnp.int32),
            pltpu.VMEM((_NBUF, _CHUNK, EMBEDDING_DIM), jnp.float32),
            [pltpu.SemaphoreType.DMA] * _NBUF,
        ],
        compiler_params=pltpu.CompilerParams(use_tc_tiling_on_sc=False),
    )
    def gather_kernel(idx_hbm, table_hbm, out_hbm, idx_v, rows_v, sems):
        wid = lax.axis_index("s") * num_cores + lax.axis_index("c")
        col0 = wid * cols_per_w
        half = col0 // _HHALF
        colq = col0 % _HHALF
        pltpu.sync_copy(idx_hbm.at[:, pl.ds(col0, cols_per_w)], idx_v)

        def fire(j, b):
            h = j // chunks_per_h
            k = j % chunks_per_h
            pltpu.async_copy(
                table_hbm.at[idx_v.at[h, pl.ds(k * _CHUNK, _CHUNK)]],
                rows_v.at[b],
                sems[b],
            )

        def drain(j, b):
            h = j // chunks_per_h
            k = j % chunks_per_h
            pltpu.make_async_copy(
                table_hbm.at[idx_v.at[h, pl.ds(k * _CHUNK, _CHUNK)]],
                rows_v.at[b],
                sems[b],
            ).wait()

        for b in range(_NBUF):
            fire(b, b)

        def body(j2, carry):
            for b in range(_NBUF):
                j = j2 * _NBUF + b
                drain(j, b)
                h = j // chunks_per_h
                k = j % chunks_per_h
                pltpu.sync_copy(
                    rows_v.at[b],
                    out_hbm.at[
                        pl.ds(h * _HHALF + colq + k * _CHUNK, _CHUNK),
                        pl.ds(half * EMBEDDING_DIM, EMBEDDING_DIM),
                    ],
                )

                @pl.when(j2 + 1 < outer)
                def _():
                    fire(j + _NBUF, b)

            return carry

        lax.fori_loop(0, outer, body, 0)

    return gather_kernel


def kernel(token_ids, embedding):
    info = plsc.get_sparse_core_info()
    num_workers = info.num_cores * info.num_subcores
    tok_t = jnp.transpose(token_ids).astype(jnp.int32)
    q = tok_t & (_TBLK - 1)
    tok_p = (tok_t - q) + ((q & (_TBLK // 2 - 1)) << 1) + (q >> 11)
    table_pairs = _table_to_row_major(jnp.transpose(embedding))
    flat_pairs = _make_gather(num_workers, info.num_cores)(
        tok_p, table_pairs.reshape(_NROWS, EMBEDDING_DIM)
    )
    out_bm = _rows_to_batch_minor(flat_pairs)
    return jnp.transpose(out_bm, (2, 0, 1))


# speedup vs baseline: 2.5684x; 1.1729x over previous
import functools

import jax
import jax.numpy as jnp
from jax import lax
from jax.experimental import pallas as pl
from jax.experimental.pallas import tpu as pltpu
from jax.experimental.pallas import tpu_sc as plsc

NUM_EMBEDDINGS = 1000000
EMBEDDING_DIM = 64
BATCH = 16384
HIST = 50

_TOTAL = BATCH * HIST
_CHUNK = 128
_NBUF = 8

_TBLK = 16384
_NTBLK = (NUM_EMBEDDINGS + _TBLK - 1) // _TBLK
_NROWS = _NTBLK * _TBLK
_HHALF = BATCH // 2


def _table_to_row_major(emb_t):
    def body(in_ref, out_ref):
        x = in_ref[...]
        out_ref[...] = jnp.concatenate(
            [x[:, :_TBLK // 2].T, x[:, _TBLK // 2:].T], axis=1
        )

    return pl.pallas_call(
        body,
        grid=(_NTBLK,),
        in_specs=[pl.BlockSpec((EMBEDDING_DIM, _TBLK), lambda i: (0, i))],
        out_specs=pl.BlockSpec((_TBLK // 2, 2 * EMBEDDING_DIM), lambda i: (i, 0)),
        out_shape=jax.ShapeDtypeStruct(
            (_NROWS // 2, 2 * EMBEDDING_DIM), jnp.float32
        ),
    )(emb_t)


def _rows_to_batch_minor(flat_pairs):
    def body(in_ref, out_ref):
        y = in_ref[...]
        out_ref[0] = jnp.concatenate(
            [y[:, :EMBEDDING_DIM].T, y[:, EMBEDDING_DIM:].T], axis=1
        )

    return pl.pallas_call(
        body,
        grid=(HIST,),
        in_specs=[pl.BlockSpec((_HHALF, 2 * EMBEDDING_DIM), lambda h: (h, 0))],
        out_specs=pl.BlockSpec((1, EMBEDDING_DIM, BATCH), lambda h: (h, 0, 0)),
        out_shape=jax.ShapeDtypeStruct((HIST, EMBEDDING_DIM, BATCH), jnp.float32),
    )(flat_pairs)


def _make_gather(num_workers: int, num_cores: int):
    cols_per_w = BATCH // num_workers
    chunks_per_h = cols_per_w // _CHUNK
    n_chunks = HIST * chunks_per_h
    outer = n_chunks // _NBUF
    mesh = plsc.VectorSubcoreMesh(core_axis_name="c", subcore_axis_name="s")

    @functools.partial(
        pl.kernel,
        mesh=mesh,
        out_type=jax.ShapeDtypeStruct((HIST * _HHALF, 2 * EMBEDDING_DIM),
                                      jnp.float32),
        scratch_types=[
            pltpu.VMEM((HIST, cols_per_w), jnp.int32),
            pltpu.VMEM((_NBUF, _CHUNK, EMBEDDING_DIM), jnp.float32),
            [pltpu.SemaphoreType.DMA] * _NBUF,
        ],
        compiler_params=pltpu.CompilerParams(use_tc_tiling_on_sc=False),
    )
    def gather_kernel(idx_hbm, table_hbm, out_hbm, idx_v, rows_v, sems):
        wid = lax.axis_index("s") * num_cores + lax.axis_index("c")
        col0 = wid * cols_per_w
        half = col0 // _HHALF
        colq = col0 % _HHALF
        pltpu.sync_copy(idx_hbm.at[:, pl.ds(col0, cols_per_w)], idx_v)

        def fire(j, b):
            h = j // chunks_per_h
            k = j % chunks_per_h
            pltpu.async_copy(
                table_hbm.at[idx_v.at[h, pl.ds(k * _CHUNK, _CHUNK)]],
                rows_v.at[b],
                sems[b],
            )

        def drain(j, b):
            h = j // chunks_per_h
            k = j % chunks_per_h
            pltpu.make_async_copy(
                table_hbm.at[idx_v.at[h, pl.ds(k * _CHUNK, _CHUNK)]],
                rows_v.at[b],
                sems[b],
            ).wait()

        for b in range(_NBUF):
            fire(b, b)

        def body(j2, carry):
            for b in range(_NBUF):
                j = j2 * _NBUF + b
                drain(j, b)
                h = j // chunks_per_h
                k = j % chunks_per_h
                pltpu.sync_copy(
                    rows_v.at[b],
                    out_hbm.at[
                        pl.ds(h * _HHALF + colq + k * _CHUNK, _CHUNK),
                        pl.ds(half * EMBEDDING_DIM, EMBEDDING_DIM),
                    ],
                )

                @pl.when(j2 + 1 < outer)
                def _():
                    fire(j + _NBUF, b)

            return carry

        lax.fori_loop(0, outer, body, 0)

    return gather_kernel


def kernel(token_ids, embedding):
    info = plsc.get_sparse_core_info()
    num_workers = info.num_cores * info.num_subcores
    tok_t = jnp.transpose(token_ids).astype(jnp.int32)
    q = tok_t & (_TBLK - 1)
    tok_p = (tok_t - q) + ((q & (_TBLK // 2 - 1)) << 1) + (q >> ((_TBLK // 2).bit_length() - 1))
    table_pairs = _table_to_row_major(jnp.transpose(embedding))
    flat_pairs = _make_gather(num_workers, info.num_cores)(
        tok_p, table_pairs.reshape(_NROWS, EMBEDDING_DIM)
    )
    out_bm = _rows_to_batch_minor(flat_pairs)
    return jnp.transpose(out_bm, (2, 0, 1))


# speedup vs baseline: 2.6849x; 1.0454x over previous
import functools

import jax
import jax.numpy as jnp
from jax import lax
from jax.experimental import pallas as pl
from jax.experimental.pallas import tpu as pltpu
from jax.experimental.pallas import tpu_sc as plsc

NUM_EMBEDDINGS = 1000000
EMBEDDING_DIM = 64
BATCH = 16384
HIST = 50

_TOTAL = BATCH * HIST
_CHUNK = 128
_NBUF = 8

_TBLK = 32768
_NTBLK = (NUM_EMBEDDINGS + _TBLK - 1) // _TBLK
_NROWS = _NTBLK * _TBLK
_HHALF = BATCH // 2


def _table_to_row_major(emb_t):
    def body(in_ref, out_ref):
        x = in_ref[...]
        out_ref[...] = jnp.concatenate(
            [x[:, :_TBLK // 2].T, x[:, _TBLK // 2:].T], axis=1
        )

    return pl.pallas_call(
        body,
        grid=(_NTBLK,),
        in_specs=[pl.BlockSpec((EMBEDDING_DIM, _TBLK), lambda i: (0, i))],
        out_specs=pl.BlockSpec((_TBLK // 2, 2 * EMBEDDING_DIM), lambda i: (i, 0)),
        out_shape=jax.ShapeDtypeStruct(
            (_NROWS // 2, 2 * EMBEDDING_DIM), jnp.float32
        ),
    )(emb_t)


def _rows_to_batch_minor(flat_pairs):
    def body(in_ref, out_ref):
        for u in range(2):
            y = in_ref[pl.ds(u * _HHALF, _HHALF), :]
            out_ref[u] = jnp.concatenate(
                [y[:, :EMBEDDING_DIM].T, y[:, EMBEDDING_DIM:].T], axis=1
            )

    return pl.pallas_call(
        body,
        grid=(HIST // 2,),
        in_specs=[pl.BlockSpec((2 * _HHALF, 2 * EMBEDDING_DIM), lambda h: (h, 0))],
        out_specs=pl.BlockSpec((2, EMBEDDING_DIM, BATCH), lambda h: (h, 0, 0)),
        out_shape=jax.ShapeDtypeStruct((HIST, EMBEDDING_DIM, BATCH), jnp.float32),
    )(flat_pairs)


def _make_gather(num_workers: int, num_cores: int):
    cols_per_w = BATCH // num_workers
    chunks_per_h = cols_per_w // _CHUNK
    n_chunks = HIST * chunks_per_h
    outer = n_chunks // _NBUF
    mesh = plsc.VectorSubcoreMesh(core_axis_name="c", subcore_axis_name="s")

    @functools.partial(
        pl.kernel,
        mesh=mesh,
        out_type=jax.ShapeDtypeStruct((HIST * _HHALF, 2 * EMBEDDING_DIM),
                                      jnp.float32),
        scratch_types=[
            pltpu.VMEM((HIST, cols_per_w), jnp.int32),
            pltpu.VMEM((_NBUF, _CHUNK, EMBEDDING_DIM), jnp.float32),
            [pltpu.SemaphoreType.DMA] * _NBUF,
        ],
        compiler_params=pltpu.CompilerParams(use_tc_tiling_on_sc=False),
    )
    def gather_kernel(idx_hbm, table_hbm, out_hbm, idx_v, rows_v, sems):
        wid = lax.axis_index("s") * num_cores + lax.axis_index("c")
        col0 = wid * cols_per_w
        half = col0 // _HHALF
        colq = col0 % _HHALF
        pltpu.sync_copy(idx_hbm.at[:, pl.ds(col0, cols_per_w)], idx_v)

        def fire(j, b):
            h = j // chunks_per_h
            k = j % chunks_per_h
            pltpu.async_copy(
                table_hbm.at[idx_v.at[h, pl.ds(k * _CHUNK, _CHUNK)]],
                rows_v.at[b],
                sems[b],
            )

        def drain(j, b):
            h = j // chunks_per_h
            k = j % chunks_per_h
            pltpu.make_async_copy(
                table_hbm.at[idx_v.at[h, pl.ds(k * _CHUNK, _CHUNK)]],
                rows_v.at[b],
                sems[b],
            ).wait()

        for b in range(_NBUF):
            fire(b, b)

        def body(j2, carry):
            for b in range(_NBUF):
                j = j2 * _NBUF + b
                drain(j, b)
                h = j // chunks_per_h
                k = j % chunks_per_h
                pltpu.sync_copy(
                    rows_v.at[b],
                    out_hbm.at[
                        pl.ds(h * _HHALF + colq + k * _CHUNK, _CHUNK),
                        pl.ds(half * EMBEDDING_DIM, EMBEDDING_DIM),
                    ],
                )

                @pl.when(j2 + 1 < outer)
                def _():
                    fire(j + _NBUF, b)

            return carry

        lax.fori_loop(0, outer, body, 0)

    return gather_kernel


def kernel(token_ids, embedding):
    info = plsc.get_sparse_core_info()
    num_workers = info.num_cores * info.num_subcores
    tok_t = jnp.transpose(token_ids).astype(jnp.int32)
    q = tok_t & (_TBLK - 1)
    tok_p = (tok_t - q) + ((q & (_TBLK // 2 - 1)) << 1) + (q >> ((_TBLK // 2).bit_length() - 1))
    table_pairs = _table_to_row_major(jnp.transpose(embedding))
    flat_pairs = _make_gather(num_workers, info.num_cores)(
        tok_p, table_pairs.reshape(_NROWS, EMBEDDING_DIM)
    )
    out_bm = _rows_to_batch_minor(flat_pairs)
    return jnp.transpose(out_bm, (2, 0, 1))
